# Initial kernel scaffold; baseline (speedup 1.0000x reference)
import jax
import jax.numpy as jnp
from jax.experimental import pallas as pl


def kernel(x_0, edge_index_0, batch_0, x_1, edge_index_1, batch_1, W1, b1, W2, b2, W3, b3, W4, b4, Wg0a, bg0a, Wg0b, bg0b, gamma0, beta0, Wg1a, bg1a, Wg1b, bg1b, gamma1, beta1, Wf0a, bf0a, Wf0b, bf0b, Wf1a, bf1a, Wf1b, bf1b):
    raise NotImplementedError("write your pallas kernel here")


# speedup vs baseline: 1.1536x; 1.1536x over previous
import functools

import jax
import jax.numpy as jnp
from jax.experimental import pallas as pl
from jax.experimental.pallas import tpu as pltpu


def _head_body(g_ref, wa_ref, ba_ref, wb_ref, bb_ref, gm_ref, bt_ref,
               wfa_ref, bfa_ref, wfb_ref, bfb_ref, g0_ref, z_ref):
    t = jnp.dot(g_ref[...], wa_ref[...], preferred_element_type=jnp.float32)
    t = jnp.maximum(t + ba_ref[...], 0.0)
    u = jnp.dot(t, wb_ref[...], preferred_element_type=jnp.float32) + bb_ref[...]
    m = jnp.mean(u, axis=0, keepdims=True)
    v = jnp.mean((u - m) ** 2, axis=0, keepdims=True)
    g0 = gm_ref[...] * (u - m) * jax.lax.rsqrt(v + 1e-5) + bt_ref[...]
    g0_ref[...] = g0
    t2 = jnp.dot(g0, wfa_ref[...], preferred_element_type=jnp.float32)
    t2 = jnp.maximum(t2 + bfa_ref[...], 0.0)
    z_ref[...] = jnp.dot(t2, wfb_ref[...], preferred_element_type=jnp.float32) + bfb_ref[...]


def _head(g, Wa, ba, Wb, bb, gamma, beta, Wfa, bfa, Wfb, bfb):
    G_, K = g.shape
    Kp = ((K + 127) // 128) * 128
    g = jnp.pad(g, ((0, 0), (0, Kp - K)))
    Wa = jnp.pad(Wa, ((0, Kp - K), (0, 0)))
    Wfb_p = jnp.pad(Wfb, ((0, 0), (0, 128 - Wfb.shape[1])))
    g0, z = pl.pallas_call(
        _head_body,
        out_shape=(
            jax.ShapeDtypeStruct((G_, 512), jnp.float32),
            jax.ShapeDtypeStruct((G_, 128), jnp.float32),
        ),
    )(g, Wa, ba.reshape(1, -1), Wb, bb.reshape(1, -1),
      gamma.reshape(1, -1), beta.reshape(1, -1),
      Wfa, bfa.reshape(1, -1), Wfb_p, jnp.pad(bfb, (0, 126)).reshape(1, -1))
    return z[:, :2], g0


_G = 512


def _gin(x, edge_index, W, b):
    src = edge_index[0]
    dst = edge_index[1]
    agg = jnp.zeros_like(x).at[dst].add(x[src])
    return (x + agg) @ W + b


def _pool(x, batch):
    s = jax.ops.segment_sum(x, batch, num_segments=_G)
    cnt = jax.ops.segment_sum(jnp.ones((x.shape[0],), x.dtype), batch,
                              num_segments=_G)
    mean = s / jnp.maximum(cnt, 1.0)[:, None]
    mx = jax.ops.segment_max(x, batch, num_segments=_G)
    mx = jnp.where(cnt[:, None] > 0, mx, 0.0)
    return jnp.concatenate([mean, mx], axis=1)


def kernel(x_0, edge_index_0, batch_0, x_1, edge_index_1, batch_1,
           W1, b1, W2, b2, W3, b3, W4, b4,
           Wg0a, bg0a, Wg0b, bg0b, gamma0, beta0,
           Wg1a, bg1a, Wg1b, bg1b, gamma1, beta1,
           Wf0a, bf0a, Wf0b, bf0b, Wf1a, bf1a, Wf1b, bf1b):
    h0 = jax.nn.relu(_gin(x_0, edge_index_0, W1, b1))
    h0 = jax.nn.relu(_gin(h0, edge_index_0, W2, b2))
    g0in = _pool(h0, batch_0)
    z, g0 = _head(g0in, Wg0a, bg0a, Wg0b, bg0b, gamma0, beta0,
                  Wf0a, bf0a, Wf0b, bf0b)
    h1 = jax.nn.relu(_gin(x_1, edge_index_1, W3, b3))
    h1 = jax.nn.relu(_gin(h1, edge_index_1, W4, b4))
    g1in = _pool(h1, batch_1)
    z1, g1 = _head(g1in, Wg1a, bg1a, Wg1b, bg1b, gamma1, beta1,
                   Wf1a, bf1a, Wf1b, bf1b)
    return (z, g0, g1, z1)


# speedup vs baseline: 4.5301x; 3.9269x over previous
import functools

import jax
import jax.numpy as jnp
from jax import lax
from jax.experimental import pallas as pl
from jax.experimental.pallas import tpu as pltpu
from jax.experimental.pallas import tpu_sc as plsc

_N = 50000
_E = 800000
_G = 512
_C = 24
_B = 2000
_NS = 16
_EPT = _E // _NS
_NBLK = _EPT // _B
_RPT = 3200
_RPT_LAST = _N - 15 * _RPT


def _sc_agg(n_b0_chunks, n_b1_chunks):
    nchunks = n_b0_chunks + n_b1_chunks
    core_of = [i % 2 for i in range(nchunks)]
    mesh = plsc.VectorSubcoreMesh(core_axis_name="c", subcore_axis_name="s")

    @functools.partial(
        pl.kernel,
        out_type=[jax.ShapeDtypeStruct((_N, _C), jnp.float32)] * nchunks,
        mesh=mesh,
        scratch_types=[
            pltpu.MemorySpace.VMEM((_B,), jnp.int32),
            pltpu.MemorySpace.VMEM((_B,), jnp.int32),
            pltpu.MemorySpace.VMEM((_B, _C), jnp.float32),
            pltpu.MemorySpace.VMEM_SHARED((_N, _C), jnp.float32),
            pltpu.SemaphoreType.DMA,
        ],
        compiler_params=pltpu.CompilerParams(use_tc_tiling_on_sc=False),
    )
    def k(*refs):
        ins = refs[:nchunks]
        src0, dst0, src1, dst1, zeros = refs[nchunks:nchunks + 5]
        outs = refs[nchunks + 5:nchunks + 5 + nchunks]
        src_v, dst_v, rows_v, acc_sh, sem = refs[nchunks + 5 + nchunks:]
        cid = lax.axis_index("c")
        sid = lax.axis_index("s")
        rstart = sid * _RPT
        e0 = sid * _EPT

        def do_chunk(u, src, dst, out):
            @pl.when(sid < 15)
            def _():
                pltpu.sync_copy(zeros.at[pl.ds(0, _RPT)],
                                acc_sh.at[pl.ds(rstart, _RPT)])

            @pl.when(sid == 15)
            def _():
                pltpu.sync_copy(zeros.at[pl.ds(0, _RPT_LAST)],
                                acc_sh.at[pl.ds(15 * _RPT, _RPT_LAST)])

            plsc.subcore_barrier()

            def blk(i, _):
                base = e0 + i * _B
                pltpu.sync_copy(src.at[pl.ds(base, _B)], src_v)
                pltpu.sync_copy(dst.at[pl.ds(base, _B)], dst_v)
                pltpu.async_copy(u.at[src_v], rows_v, sem).wait()
                pltpu.sync_copy(rows_v, acc_sh.at[dst_v], add=True)
                return ()

            lax.fori_loop(0, _NBLK, blk, ())
            plsc.subcore_barrier()

            @pl.when(sid < 15)
            def _():
                pltpu.sync_copy(acc_sh.at[pl.ds(rstart, _RPT)],
                                out.at[pl.ds(rstart, _RPT)])

            @pl.when(sid == 15)
            def _():
                pltpu.sync_copy(acc_sh.at[pl.ds(15 * _RPT, _RPT_LAST)],
                                out.at[pl.ds(15 * _RPT, _RPT_LAST)])

        for ci in range(nchunks):
            src, dst = (src0, dst0) if ci < n_b0_chunks else (src1, dst1)

            @pl.when(cid == core_of[ci])
            def _(ci=ci, src=src, dst=dst):
                do_chunk(ins[ci], src, dst, outs[ci])

    return k


def _agg_layer(x0_chunks, x1_chunks, e0, e1, zeros):
    k = _sc_agg(len(x0_chunks), len(x1_chunks))
    outs = k(*x0_chunks, *x1_chunks, e0[0], e0[1], e1[0], e1[1], zeros)
    return outs[:len(x0_chunks)], outs[len(x0_chunks):]


def _gin_mm_body(x_ref, a_ref, w_ref, b_ref, o_ref):
    s = x_ref[...] + a_ref[...]
    y = jnp.dot(s, w_ref[...], preferred_element_type=jnp.float32)
    o_ref[...] = jnp.maximum(y + b_ref[...], 0.0)


def _gin_mm(x, a, W, b):
    N, K = x.shape
    M = W.shape[1]
    BM = 2000
    grid = (N // BM,)
    return pl.pallas_call(
        _gin_mm_body,
        grid=grid,
        in_specs=[
            pl.BlockSpec((BM, K), lambda i: (i, 0)),
            pl.BlockSpec((BM, K), lambda i: (i, 0)),
            pl.BlockSpec((K, M), lambda i: (0, 0)),
            pl.BlockSpec((1, M), lambda i: (0, 0)),
        ],
        out_specs=pl.BlockSpec((BM, M), lambda i: (i, 0)),
        out_shape=jax.ShapeDtypeStruct((N, M), jnp.float32),
    )(x, a, W, b.reshape(1, -1))


def _head_body(g_ref, wa_ref, ba_ref, wb_ref, bb_ref, gm_ref, bt_ref,
               wfa_ref, bfa_ref, wfb_ref, bfb_ref, g0_ref, z_ref):
    t = jnp.dot(g_ref[...], wa_ref[...], preferred_element_type=jnp.float32)
    t = jnp.maximum(t + ba_ref[...], 0.0)
    u = jnp.dot(t, wb_ref[...], preferred_element_type=jnp.float32) + bb_ref[...]
    m = jnp.mean(u, axis=0, keepdims=True)
    v = jnp.mean((u - m) ** 2, axis=0, keepdims=True)
    g0 = gm_ref[...] * (u - m) * lax.rsqrt(v + 1e-5) + bt_ref[...]
    g0_ref[...] = g0
    t2 = jnp.dot(g0, wfa_ref[...], preferred_element_type=jnp.float32)
    t2 = jnp.maximum(t2 + bfa_ref[...], 0.0)
    z_ref[...] = jnp.dot(t2, wfb_ref[...], preferred_element_type=jnp.float32) + bfb_ref[...]


def _head(g, Wa, ba, Wb, bb, gamma, beta, Wfa, bfa, Wfb, bfb):
    G_, K = g.shape
    Kp = ((K + 127) // 128) * 128
    g = jnp.pad(g, ((0, 0), (0, Kp - K)))
    Wa = jnp.pad(Wa, ((0, Kp - K), (0, 0)))
    Wfb_p = jnp.pad(Wfb, ((0, 0), (0, 128 - Wfb.shape[1])))
    g0, z = pl.pallas_call(
        _head_body,
        out_shape=(
            jax.ShapeDtypeStruct((G_, 512), jnp.float32),
            jax.ShapeDtypeStruct((G_, 128), jnp.float32),
        ),
    )(g, Wa, ba.reshape(1, -1), Wb, bb.reshape(1, -1),
      gamma.reshape(1, -1), beta.reshape(1, -1),
      Wfa, bfa.reshape(1, -1), Wfb_p, jnp.pad(bfb, (0, 126)).reshape(1, -1))
    return z[:, :2], g0


def _chunks(x, n):
    return [lax.slice(x, (0, i * _C), (_N, (i + 1) * _C)) for i in range(n)]


def _pool_xla(x, batch, d_real):
    s = jax.ops.segment_sum(x, batch, num_segments=_G)
    cnt = jax.ops.segment_sum(jnp.ones((x.shape[0],), x.dtype), batch,
                              num_segments=_G)
    mean = s / jnp.maximum(cnt, 1.0)[:, None]
    mx = jax.ops.segment_max(x, batch, num_segments=_G)
    mx = jnp.where(cnt[:, None] > 0, mx, 0.0)
    return jnp.concatenate([mean[:, :d_real], mx[:, :d_real]], axis=1)


def kernel(x_0, edge_index_0, batch_0, x_1, edge_index_1, batch_1,
           W1, b1, W2, b2, W3, b3, W4, b4,
           Wg0a, bg0a, Wg0b, bg0b, gamma0, beta0,
           Wg1a, bg1a, Wg1b, bg1b, gamma1, beta1,
           Wf0a, bf0a, Wf0b, bf0b, Wf1a, bf1a, Wf1b, bf1b):
    f32 = jnp.float32
    D0p, D1p = 96, 48
    M0p, M1p = 1024, 512

    x0p = jnp.pad(x_0, ((0, 0), (0, D0p - x_0.shape[1])))
    x1p = jnp.pad(x_1, ((0, 0), (0, D1p - x_1.shape[1])))
    W1p = jnp.pad(W1, ((0, D0p - 93), (0, D0p - 93)))
    b1p = jnp.pad(b1, (0, D0p - 93))
    W2p = jnp.pad(W2, ((0, D0p - 93), (0, M0p - 930)))
    b2p = jnp.pad(b2, (0, M0p - 930))
    W3p = jnp.pad(W3, ((0, D1p - 43), (0, D1p - 43)))
    b3p = jnp.pad(b3, (0, D1p - 43))
    W4p = jnp.pad(W4, ((0, D1p - 43), (0, M1p - 430)))
    b4p = jnp.pad(b4, (0, M1p - 430))
    zeros = jnp.zeros((_RPT, _C), f32)

    a0c, a1c = _agg_layer(_chunks(x0p, 4), _chunks(x1p, 2),
                          edge_index_0, edge_index_1, zeros)
    agg_x0 = jnp.concatenate(a0c, axis=1)
    agg_x1 = jnp.concatenate(a1c, axis=1)

    h0 = _gin_mm(x0p, agg_x0, W1p, b1p)
    h1 = _gin_mm(x1p, agg_x1, W3p, b3p)

    b0c, b1c = _agg_layer(_chunks(h0, 4), _chunks(h1, 2),
                          edge_index_0, edge_index_1, zeros)
    agg_h0 = jnp.concatenate(b0c, axis=1)
    agg_h1 = jnp.concatenate(b1c, axis=1)

    h2_0 = _gin_mm(h0, agg_h0, W2p, b2p)
    h2_1 = _gin_mm(h1, agg_h1, W4p, b4p)

    g0in = _pool_xla(h2_0, batch_0, 930)
    g1in = _pool_xla(h2_1, batch_1, 430)

    z, g0 = _head(g0in, Wg0a, bg0a, Wg0b, bg0b, gamma0, beta0,
                  Wf0a, bf0a, Wf0b, bf0b)
    z1, g1 = _head(g1in, Wg1a, bg1a, Wg1b, bg1b, gamma1, beta1,
                   Wf1a, bf1a, Wf1b, bf1b)
    return (z, g0, g1, z1)


# speedup vs baseline: 5.3926x; 1.1904x over previous
import functools

import jax
import jax.numpy as jnp
from jax import lax
from jax.experimental import pallas as pl
from jax.experimental.pallas import tpu as pltpu
from jax.experimental.pallas import tpu_sc as plsc

_N = 50000
_E = 800000
_G = 512
_C = 24
_B = 2000
_NS = 16
_EPT = _E // _NS
_NBLK = _EPT // _B
_RPT = 3200
_RPT_LAST = _N - 15 * _RPT


def _sc_agg(nchunks):
    core_of = [i % 2 for i in range(nchunks)]
    mesh = plsc.VectorSubcoreMesh(core_axis_name="c", subcore_axis_name="s")

    @functools.partial(
        pl.kernel,
        out_type=[jax.ShapeDtypeStruct((_N, _C), jnp.float32)] * nchunks,
        mesh=mesh,
        scratch_types=[
            pltpu.MemorySpace.VMEM((_B,), jnp.int32),
            pltpu.MemorySpace.VMEM((_B,), jnp.int32),
            pltpu.MemorySpace.VMEM((_B, _C), jnp.float32),
            pltpu.MemorySpace.VMEM_SHARED((_N, _C), jnp.float32),
            pltpu.SemaphoreType.DMA,
        ],
        compiler_params=pltpu.CompilerParams(use_tc_tiling_on_sc=False),
    )
    def k(*refs):
        ins = refs[:nchunks]
        src, dst, zeros = refs[nchunks:nchunks + 3]
        outs = refs[nchunks + 3:nchunks + 3 + nchunks]
        src_v, dst_v, rows_v, acc_sh, sem = refs[nchunks + 3 + nchunks:]
        cid = lax.axis_index("c")
        sid = lax.axis_index("s")
        rstart = sid * _RPT
        e0 = sid * _EPT

        def do_chunk(u, src, dst, out):
            @pl.when(sid < 15)
            def _():
                pltpu.sync_copy(zeros.at[pl.ds(0, _RPT)],
                                acc_sh.at[pl.ds(rstart, _RPT)])

            @pl.when(sid == 15)
            def _():
                pltpu.sync_copy(zeros.at[pl.ds(0, _RPT_LAST)],
                                acc_sh.at[pl.ds(15 * _RPT, _RPT_LAST)])

            plsc.subcore_barrier()

            def blk(i, _):
                base = e0 + i * _B
                pltpu.sync_copy(src.at[pl.ds(base, _B)], src_v)
                pltpu.sync_copy(dst.at[pl.ds(base, _B)], dst_v)
                pltpu.async_copy(u.at[src_v], rows_v, sem).wait()
                pltpu.sync_copy(rows_v, acc_sh.at[dst_v], add=True)
                return ()

            lax.fori_loop(0, _NBLK, blk, ())
            plsc.subcore_barrier()

            @pl.when(sid < 15)
            def _():
                pltpu.sync_copy(acc_sh.at[pl.ds(rstart, _RPT)],
                                out.at[pl.ds(rstart, _RPT)])

            @pl.when(sid == 15)
            def _():
                pltpu.sync_copy(acc_sh.at[pl.ds(15 * _RPT, _RPT_LAST)],
                                out.at[pl.ds(15 * _RPT, _RPT_LAST)])

        for ci in range(nchunks):
            @pl.when(cid == core_of[ci])
            def _(ci=ci):
                do_chunk(ins[ci], src, dst, outs[ci])

    return k


def _agg(x_chunks, edge_index, zeros):
    k = _sc_agg(len(x_chunks))
    return k(*x_chunks, edge_index[0], edge_index[1], zeros)


def _gin_mm_body(x_ref, a_ref, w_ref, b_ref, o_ref):
    s = x_ref[...] + a_ref[...]
    y = jnp.dot(s, w_ref[...], preferred_element_type=jnp.float32)
    o_ref[...] = jnp.maximum(y + b_ref[...], 0.0)


def _gin_mm(x, a, W, b):
    N, K = x.shape
    M = W.shape[1]
    BM = 2000
    grid = (N // BM,)
    return pl.pallas_call(
        _gin_mm_body,
        grid=grid,
        in_specs=[
            pl.BlockSpec((BM, K), lambda i: (i, 0)),
            pl.BlockSpec((BM, K), lambda i: (i, 0)),
            pl.BlockSpec((K, M), lambda i: (0, 0)),
            pl.BlockSpec((1, M), lambda i: (0, 0)),
        ],
        out_specs=pl.BlockSpec((BM, M), lambda i: (i, 0)),
        out_shape=jax.ShapeDtypeStruct((N, M), jnp.float32),
    )(x, a, W, b.reshape(1, -1))


def _gin2_pool_body(h_ref, a_ref, w_ref, b_ref, seg_ref,
                    sum_ref, max_ref, cnt_ref, *, bm):
    y = jnp.dot(h_ref[...] + a_ref[...], w_ref[...],
                preferred_element_type=jnp.float32)
    y = jnp.maximum(y + b_ref[...], 0.0)
    seg = seg_ref[...]
    p = (lax.broadcasted_iota(jnp.int32, (bm, _G), 1) == seg
         ).astype(jnp.float32)
    psum = lax.dot_general(p, y, (((0,), (0,)), ((), ())),
                           precision=lax.Precision.HIGHEST,
                           preferred_element_type=jnp.float32)
    pcnt = lax.dot_general(p, jnp.ones((bm, 1), jnp.float32),
                           (((0,), (0,)), ((), ())),
                           precision=lax.Precision.HIGHEST,
                           preferred_element_type=jnp.float32)
    m = y
    k = 1
    while k < bm:
        m_sh = jnp.concatenate([m[-k:], m[:-k]], axis=0)
        same = seg == jnp.concatenate([seg[-k:], seg[:-k]], axis=0)
        m = jnp.maximum(m, jnp.where(same, m_sh, 0.0))
        k *= 2
    nxt = jnp.concatenate([seg[1:], seg[-1:] + 1], axis=0)
    bmask = (seg != nxt).astype(jnp.float32)
    pick = lax.dot_general(p * bmask, m, (((0,), (0,)), ((), ())),
                           precision=lax.Precision.HIGHEST,
                           preferred_element_type=jnp.float32)

    @pl.when(pl.program_id(0) == 0)
    def _():
        sum_ref[...] = psum
        max_ref[...] = pick
        cnt_ref[...] = pcnt

    @pl.when(pl.program_id(0) != 0)
    def _():
        sum_ref[...] += psum
        max_ref[...] = jnp.maximum(max_ref[...], pick)
        cnt_ref[...] += pcnt


def _gin2_pool(h, a, W, b, seg_col):
    N, K = h.shape
    M = W.shape[1]
    BM = 1000
    return pl.pallas_call(
        functools.partial(_gin2_pool_body, bm=BM),
        grid=(N // BM,),
        in_specs=[
            pl.BlockSpec((BM, K), lambda i: (i, 0)),
            pl.BlockSpec((BM, K), lambda i: (i, 0)),
            pl.BlockSpec((K, M), lambda i: (0, 0)),
            pl.BlockSpec((1, M), lambda i: (0, 0)),
            pl.BlockSpec((BM, 1), lambda i: (i, 0)),
        ],
        out_specs=(
            pl.BlockSpec((_G, M), lambda i: (0, 0)),
            pl.BlockSpec((_G, M), lambda i: (0, 0)),
            pl.BlockSpec((_G, 1), lambda i: (0, 0)),
        ),
        out_shape=(
            jax.ShapeDtypeStruct((_G, M), jnp.float32),
            jax.ShapeDtypeStruct((_G, M), jnp.float32),
            jax.ShapeDtypeStruct((_G, 1), jnp.float32),
        ),
    )(h, a, W, b.reshape(1, -1), seg_col)


def _head_body(s_ref, mx_ref, cnt_ref, wam_ref, wax_ref, ba_ref,
               wb_ref, bb_ref, gm_ref, bt_ref,
               wfa_ref, bfa_ref, wfb_ref, bfb_ref, g0_ref, z_ref):
    inv = 1.0 / jnp.maximum(cnt_ref[...], 1.0)
    mean = s_ref[...] * inv
    t = (jnp.dot(mean, wam_ref[...], preferred_element_type=jnp.float32)
         + jnp.dot(mx_ref[...], wax_ref[...], preferred_element_type=jnp.float32))
    t = jnp.maximum(t + ba_ref[...], 0.0)
    u = jnp.dot(t, wb_ref[...], preferred_element_type=jnp.float32) + bb_ref[...]
    m = jnp.mean(u, axis=0, keepdims=True)
    v = jnp.mean((u - m) ** 2, axis=0, keepdims=True)
    g0 = gm_ref[...] * (u - m) * lax.rsqrt(v + 1e-5) + bt_ref[...]
    g0_ref[...] = g0
    t2 = jnp.dot(g0, wfa_ref[...], preferred_element_type=jnp.float32)
    t2 = jnp.maximum(t2 + bfa_ref[...], 0.0)
    z_ref[...] = jnp.dot(t2, wfb_ref[...], preferred_element_type=jnp.float32) + bfb_ref[...]


def _head(psum, pmax, cnt, d_real, Wa, ba, Wb, bb, gamma, beta,
          Wfa, bfa, Wfb, bfb):
    Mp = psum.shape[1]
    Wam = jnp.pad(Wa[:d_real], ((0, Mp - d_real), (0, 0)))
    Wax = jnp.pad(Wa[d_real:], ((0, Mp - d_real), (0, 0)))
    Wfb_p = jnp.pad(Wfb, ((0, 0), (0, 128 - Wfb.shape[1])))
    g0, z = pl.pallas_call(
        _head_body,
        out_shape=(
            jax.ShapeDtypeStruct((_G, 512), jnp.float32),
            jax.ShapeDtypeStruct((_G, 128), jnp.float32),
        ),
    )(psum, pmax, cnt, Wam, Wax, ba.reshape(1, -1), Wb, bb.reshape(1, -1),
      gamma.reshape(1, -1), beta.reshape(1, -1),
      Wfa, bfa.reshape(1, -1), Wfb_p, jnp.pad(bfb, (0, 126)).reshape(1, -1))
    return z[:, :2], g0


def _chunks(x, n):
    return [lax.slice(x, (0, i * _C), (_N, (i + 1) * _C)) for i in range(n)]


def kernel(x_0, edge_index_0, batch_0, x_1, edge_index_1, batch_1,
           W1, b1, W2, b2, W3, b3, W4, b4,
           Wg0a, bg0a, Wg0b, bg0b, gamma0, beta0,
           Wg1a, bg1a, Wg1b, bg1b, gamma1, beta1,
           Wf0a, bf0a, Wf0b, bf0b, Wf1a, bf1a, Wf1b, bf1b):
    f32 = jnp.float32
    D0p, D1p = 96, 48
    M0p, M1p = 1024, 512

    x0p = jnp.pad(x_0, ((0, 0), (0, D0p - x_0.shape[1])))
    x1p = jnp.pad(x_1, ((0, 0), (0, D1p - x_1.shape[1])))
    W1p = jnp.pad(W1, ((0, D0p - 93), (0, D0p - 93)))
    b1p = jnp.pad(b1, (0, D0p - 93))
    W2p = jnp.pad(W2, ((0, D0p - 93), (0, M0p - 930)))
    b2p = jnp.pad(b2, (0, M0p - 930))
    W3p = jnp.pad(W3, ((0, D1p - 43), (0, D1p - 43)))
    b3p = jnp.pad(b3, (0, D1p - 43))
    W4p = jnp.pad(W4, ((0, D1p - 43), (0, M1p - 430)))
    b4p = jnp.pad(b4, (0, M1p - 430))
    zeros = jnp.zeros((_RPT, _C), f32)
    seg0 = batch_0.reshape(-1, 1)
    seg1 = batch_1.reshape(-1, 1)

    a0c = _agg(_chunks(x0p, 4), edge_index_0, zeros)
    a1c = _agg(_chunks(x1p, 2), edge_index_1, zeros)
    h0 = _gin_mm(x0p, jnp.concatenate(a0c, axis=1), W1p, b1p)
    h1 = _gin_mm(x1p, jnp.concatenate(a1c, axis=1), W3p, b3p)
    b0c = _agg(_chunks(h0, 4), edge_index_0, zeros)
    b1c = _agg(_chunks(h1, 2), edge_index_1, zeros)
    s0, m0, c0 = _gin2_pool(h0, jnp.concatenate(b0c, axis=1), W2p, b2p, seg0)
    s1, m1, c1 = _gin2_pool(h1, jnp.concatenate(b1c, axis=1), W4p, b4p, seg1)

    z, g0 = _head(s0, m0, c0, 930, Wg0a, bg0a, Wg0b, bg0b, gamma0, beta0,
                  Wf0a, bf0a, Wf0b, bf0b)
    z1, g1 = _head(s1, m1, c1, 430, Wg1a, bg1a, Wg1b, bg1b, gamma1, beta1,
                   Wf1a, bf1a, Wf1b, bf1b)
    return (z, g0, g1, z1)
